# Initial kernel scaffold; baseline (speedup 1.0000x reference)
import jax
import jax.numpy as jnp
from jax.experimental import pallas as pl


def kernel(lambda_A, lambda_B, lambda_Pi, lambda_SP, pos, x, leaves, batch, levels_pa, levels_ch):
    raise NotImplementedError("write your pallas kernel here")


# speedup vs baseline: 7.8654x; 7.8654x over previous
import numpy as np
import jax
import jax.numpy as jnp
from jax.experimental import pallas as pl

_N_TREES = 8
_DEPTH = 12
_C = 16
_L = 2
_M = 512
_N_GEN = 16
_T = 2 ** (_DEPTH + 1) - 1
_F = _C * _N_GEN
_NLEAF = 2 ** _DEPTH


def _bitrev(j: int, bits: int) -> int:
    r = 0
    for _ in range(bits):
        r = (r << 1) | (j & 1)
        j >>= 1
    return r


def _build_x_index() -> np.ndarray:
    idx = np.zeros((_N_TREES, _NLEAF, _DEPTH + 1), dtype=np.int32)
    for d in range(_DEPTH + 1):
        n = 1 << d
        src = (n - 1) + np.array([_bitrev(j, d) for j in range(n)], dtype=np.int64)
        for t in range(_N_TREES):
            idx[t, :n, d] = (t * _T + src).astype(np.int32)
    return idx


_X_IDX = _build_x_index()


def _group_sum_mat(dtype):
    grp = jax.lax.broadcasted_iota(jnp.int32, (_F, _C), 0) // _C
    col = jax.lax.broadcasted_iota(jnp.int32, (_F, _C), 1)
    return (grp == col).astype(dtype)


def _prep_body(lam_a_ref, lam_b_ref, lam_pi_ref, lam_sp_ref,
               w_ref, b_ref, leaf0_ref, leaf1_ref, ll0_ref, ll1_ref):
    f32 = jnp.float32
    lb = lam_b_ref[...]
    mb = jnp.max(lb, axis=0, keepdims=True)
    eb = jnp.exp(lb - mb)
    b_sm = eb / jnp.sum(eb, axis=0, keepdims=True)
    b_ref[...] = b_sm

    la = lam_a_ref[...]
    ma = jnp.max(la, axis=1, keepdims=True)
    ea = jnp.exp(la - ma)
    a_sm = ea / jnp.sum(ea, axis=1, keepdims=True)

    lp = lam_pi_ref[...]
    mp = jnp.max(lp, axis=1, keepdims=True)
    ep = jnp.exp(lp - mp)
    pi_sm = ep / jnp.sum(ep, axis=1, keepdims=True)

    ls = lam_sp_ref[...]
    ms = jnp.max(ls, axis=0, keepdims=True)
    es = jnp.exp(ls - ms)
    sp = es / jnp.sum(es, axis=0, keepdims=True)

    w_ref[...] = jnp.zeros((2 * _F, _F), f32)
    pi_rows = [None, None]
    for l in range(_L):
        row = jnp.zeros((1, _F), f32)
        for g in range(_C):
            r0 = l * _F + g * _C
            blk = a_sm[r0:r0 + _C, 0:_C] * sp[l:l + 1, g:g + 1]
            w_ref[r0:r0 + _C, g * _C:(g + 1) * _C] = blk
        parts = [pi_sm[l * _C + g:l * _C + g + 1, 0:_C] for g in range(_C)]
        pi_rows[l] = jnp.concatenate(parts, axis=1)

    s_mat = _group_sum_mat(f32)
    s_mat_t = jnp.transpose(s_mat)
    for l, (leaf_ref, ll_ref) in enumerate(((leaf0_ref, ll0_ref), (leaf1_ref, ll1_ref))):
        pb = b_sm * pi_rows[l]
        nu = jax.lax.dot_general(pb, s_mat, (((1,), (0,)), ((), ())),
                                 preferred_element_type=f32)
        inv = jax.lax.dot_general(1.0 / nu, s_mat_t, (((1,), (0,)), ((), ())),
                                  preferred_element_type=f32)
        leaf_ref[...] = pb * inv
        ll_ref[...] = jnp.log(nu)


def _sweep_body(x_ref, w_ref, b_ref, leaf0_ref, leaf1_ref, ll0_ref, ll1_ref, out_ref):
    f32 = jnp.float32
    bf16 = jnp.bfloat16
    xv = x_ref[0]
    w_b = w_ref[...].astype(bf16)
    b_b = b_ref[...].astype(bf16)
    w0 = w_b[0:_F, :]
    w1 = w_b[_F:2 * _F, :]
    s_mat_b = _group_sum_mat(bf16)
    s_mat_t_b = jnp.transpose(s_mat_b)

    def onehot(n, d, dtype):
        xcol = jax.lax.slice(xv, (0, d), (n, d + 1))
        iot = jax.lax.broadcasted_iota(jnp.int32, (n, _M), 1)
        return (xcol == iot).astype(dtype)

    oh = onehot(_NLEAF, _DEPTH, bf16)
    half = _NLEAF // 2
    l0 = leaf0_ref[...].astype(bf16)
    l1 = leaf1_ref[...].astype(bf16)
    dn = (((1,), (0,)), ((), ()))
    beta = jnp.concatenate(
        [jax.lax.dot_general(oh[0:half], l0, dn, preferred_element_type=f32),
         jax.lax.dot_general(oh[half:_NLEAF], l1, dn, preferred_element_type=f32)],
        axis=0).astype(bf16)
    ohf = oh.astype(f32)
    cnt0 = jnp.sum(ohf[0:half], axis=0, keepdims=True)
    cnt1 = jnp.sum(ohf[half:_NLEAF], axis=0, keepdims=True)
    ll_acc = (jax.lax.dot_general(cnt0, ll0_ref[...], dn, preferred_element_type=f32) +
              jax.lax.dot_general(cnt1, ll1_ref[...], dn, preferred_element_type=f32))

    for d in range(_DEPTH - 1, -1, -1):
        n = 1 << d
        t_pa = (jax.lax.dot_general(beta[0:n], w0, dn, preferred_element_type=f32) +
                jax.lax.dot_general(beta[n:2 * n], w1, dn, preferred_element_type=f32))
        bvals = jax.lax.dot_general(onehot(n, d, bf16), b_b, dn,
                                    preferred_element_type=f32)
        u = t_pa * bvals
        u_b = u.astype(bf16)
        nu = jax.lax.dot_general(u_b, s_mat_b, dn, preferred_element_type=f32)
        ll_acc = ll_acc + jnp.sum(jnp.log(nu), axis=0, keepdims=True)
        inv = jax.lax.dot_general((1.0 / nu).astype(bf16), s_mat_t_b, dn,
                                  preferred_element_type=f32)
        beta = (u * inv).astype(bf16)

    out_ref[...] = ll_acc.reshape(1, 1, _N_GEN)


def kernel(lambda_A, lambda_B, lambda_Pi, lambda_SP, pos, x, leaves, batch, levels_pa, levels_ch):
    f32 = jnp.float32
    lam_a = jnp.transpose(lambda_A, (2, 3, 1, 0)).reshape(2 * _F, _C).astype(f32)
    lam_b = jnp.transpose(lambda_B, (1, 2, 0)).reshape(_M, _F).astype(f32)
    lam_pi = jnp.transpose(lambda_Pi, (1, 2, 0)).reshape(_L * _C, _C).astype(f32)
    lam_sp = lambda_SP.astype(f32)
    xg = jnp.asarray(x).astype(jnp.int32)[_X_IDX]

    w_mat, b_mat, leaf0, leaf1, ll0, ll1 = pl.pallas_call(
        _prep_body,
        out_shape=(
            jax.ShapeDtypeStruct((2 * _F, _F), f32),
            jax.ShapeDtypeStruct((_M, _F), f32),
            jax.ShapeDtypeStruct((_M, _F), f32),
            jax.ShapeDtypeStruct((_M, _F), f32),
            jax.ShapeDtypeStruct((_M, _N_GEN), f32),
            jax.ShapeDtypeStruct((_M, _N_GEN), f32),
        ),
    )(lam_a, lam_b, lam_pi, lam_sp)

    out = pl.pallas_call(
        _sweep_body,
        grid=(_N_TREES,),
        in_specs=[
            pl.BlockSpec((1, _NLEAF, _DEPTH + 1), lambda t: (t, 0, 0)),
            pl.BlockSpec((2 * _F, _F), lambda t: (0, 0)),
            pl.BlockSpec((_M, _F), lambda t: (0, 0)),
            pl.BlockSpec((_M, _F), lambda t: (0, 0)),
            pl.BlockSpec((_M, _F), lambda t: (0, 0)),
            pl.BlockSpec((_M, _N_GEN), lambda t: (0, 0)),
            pl.BlockSpec((_M, _N_GEN), lambda t: (0, 0)),
        ],
        out_specs=pl.BlockSpec((1, 1, _N_GEN), lambda t: (t, 0, 0)),
        out_shape=jax.ShapeDtypeStruct((_N_TREES, 1, _N_GEN), f32),
    )(xg, w_mat, b_mat, leaf0, leaf1, ll0, ll1)
    return out.reshape(_N_TREES, _N_GEN)
